# baseline (speedup 1.0000x reference)
import functools

import jax
import jax.numpy as jnp
from jax import lax
from jax.experimental import pallas as pl
from jax.experimental.pallas import tpu as pltpu
from jax.experimental.pallas import tpu_sc as plsc

B = 16384
EMB = 64
QLEN = 20
ENT = 1000000
WORD = 100000
NC = 2
NS = 16
NW = NC * NS
BPW = B // NW
CB = 16
NCHUNK = BPW // CB


RBLK = 2048


def _pair_row(g):
    return (g >> 11) * (RBLK // 2) + (g & (RBLK // 2 - 1))


def _pair_off(g):
    return ((g >> 10) & 1) * EMB


def _repack_body(x_ref, eye_ref, out_ref):
    x = x_ref[...]
    y = lax.dot_general(x, eye_ref[...], (((0,), (0,)), ((), ())),
                        precision=lax.Precision.HIGHEST,
                        preferred_element_type=jnp.float32)
    out_ref[...] = jnp.concatenate(
        [y[:RBLK // 2], y[RBLK // 2:]], axis=1)


def _repack(table_t, n):
    grid = (n + RBLK - 1) // RBLK
    eye = jnp.eye(EMB, dtype=jnp.float32)
    return pl.pallas_call(
        _repack_body,
        grid=(grid,),
        in_specs=[
            pl.BlockSpec((EMB, RBLK), lambda i: (0, i)),
            pl.BlockSpec((EMB, EMB), lambda i: (0, 0)),
        ],
        out_specs=pl.BlockSpec((RBLK // 2, 2 * EMB), lambda i: (i, 0)),
        out_shape=jax.ShapeDtypeStruct((grid * RBLK // 2, 2 * EMB),
                                       jnp.float32),
    )(table_t, eye)


def _qw_prep_body(qw_ref, idx_ref, off_ref):
    pad = jnp.zeros((24 - QLEN, 128), jnp.int32)
    xf = lax.bitcast_convert_type(qw_ref[...], jnp.float32)
    xt = lax.bitcast_convert_type(jnp.transpose(xf), jnp.int32)
    row = _pair_row(xt)
    off = _pair_off(xt)
    for s in range(8):
        idx_ref[:, s, :] = jnp.concatenate(
            [row[:, 128 * s:128 * (s + 1)], pad], axis=0)
        off_ref[:, s, :] = jnp.concatenate(
            [off[:, 128 * s:128 * (s + 1)], pad], axis=0)


def _qw_prep(query_words):
    return pl.pallas_call(
        _qw_prep_body,
        grid=(B // 1024,),
        in_specs=[pl.BlockSpec((1024, QLEN), lambda i: (i, 0))],
        out_specs=(
            pl.BlockSpec((24, 8, 128), lambda i: (0, i, 0)),
            pl.BlockSpec((24, 8, 128), lambda i: (0, i, 0)),
        ),
        out_shape=(
            jax.ShapeDtypeStruct((24, B // 128, 128), jnp.int32),
            jax.ShapeDtypeStruct((24, B // 128, 128), jnp.int32),
        ),
    )(query_words)


def _sc_word_body(qwp_hbm, qwo_hbm, word_hbm, qsum_out,
                  widx, woff, wrows, qbuf, woff_s, sem0, sem1):
    wid = lax.axis_index("s") * NC + lax.axis_index("c")
    orow = (wid % (NW // 2)) * BPW
    ocol = (wid // (NW // 2)) * EMB
    sems = (sem0, sem1)

    pltpu.sync_copy(
        qwp_hbm.at[pl.ds(0, 24), pl.ds(wid * 4, 4)], widx)
    pltpu.sync_copy(
        qwo_hbm.at[pl.ds(0, 24), pl.ds(wid * 4, 4)], woff)

    def idx_slice(w, c):
        return widx.at[w, c // 8, pl.ds((c % 8) * CB, CB)]

    def fire(c, p):
        sem = sems[p]
        for w in range(QLEN):
            pltpu.async_copy(word_hbm.at[idx_slice(w, c)],
                             wrows.at[p].at[w], sem)

    def drain(c, p):
        sem = sems[p]
        for w in range(QLEN):
            pltpu.make_async_copy(word_hbm.at[idx_slice(w, c)],
                                  wrows.at[p].at[w], sem).wait()

    def compute(c, p):
        wr = wrows.at[p]
        qb = qbuf.at[p]
        ws = woff_s.at[p]
        for w in range(QLEN):
            wv = woff[w, c // 8, pl.ds((c % 8) * CB, CB)]
            for i in range(CB):
                ws[w, i] = wv[i]

        def elem_body(i, _):
            accs = [None] * (EMB // 16)
            for w in range(QLEN):
                off_w = ws[w, i]
                for j in range(EMB // 16):
                    v = wr[w, i, pl.ds(off_w + 16 * j, 16)]
                    accs[j] = v if w == 0 else accs[j] + v
            for j in range(EMB // 16):
                qb[i, pl.ds(16 * j, 16)] = accs[j]
            return 0

        lax.fori_loop(0, CB, elem_body, 0)
        pltpu.sync_copy(
            qb, qsum_out.at[pl.ds(orow + c * CB, CB), pl.ds(ocol, EMB)])

    fire(0, 0)

    def pair_body(t, _):
        c0 = 2 * t
        fire(c0 + 1, 1)
        drain(c0, 0)
        compute(c0, 0)

        @pl.when(t < NCHUNK // 2 - 1)
        def _():
            fire(c0 + 2, 0)

        drain(c0 + 1, 1)
        compute(c0 + 1, 1)
        return 0

    lax.fori_loop(0, NCHUNK // 2, pair_body, 0)


_sc_word = functools.partial(
    pl.kernel,
    out_type=jax.ShapeDtypeStruct((B // 2, 2 * EMB), jnp.float32),
    mesh=plsc.VectorSubcoreMesh(core_axis_name="c", subcore_axis_name="s"),
    compiler_params=pltpu.CompilerParams(use_tc_tiling_on_sc=False),
    scratch_types=[
        pltpu.VMEM((24, 4, 128), jnp.int32),
        pltpu.VMEM((24, 4, 128), jnp.int32),
        pltpu.VMEM((2, QLEN, CB, 2 * EMB), jnp.float32),
        pltpu.VMEM((2, CB, EMB), jnp.float32),
        pltpu.SMEM((2, 24, CB), jnp.int32),
        pltpu.SemaphoreType.DMA,
        pltpu.SemaphoreType.DMA,
    ],
)(_sc_word_body)


def _sc_user_body(up_hbm, uo_hbm, ent_hbm, user_out,
                  uidx, uoff, ubuf, uout, uoff_s, sem0):
    wid = lax.axis_index("s") * NC + lax.axis_index("c")
    base = wid * BPW
    orow = (wid % (NW // 2)) * BPW
    ocol = (wid // (NW // 2)) * EMB

    pltpu.sync_copy(up_hbm.at[pl.ds(base, BPW)], uidx)
    pltpu.sync_copy(uo_hbm.at[pl.ds(base, BPW)], uoff)

    cps = [pltpu.async_copy(ent_hbm.at[uidx.at[pl.ds(k * 128, 128)]],
                            ubuf.at[pl.ds(k * 128, 128)], sem0)
           for k in range(BPW // 128)]

    for cc in range(BPW // 16):
        uv = uoff[pl.ds(cc * 16, 16)]
        for i in range(16):
            uoff_s[cc * 16 + i] = uv[i]

    for cp in cps:
        cp.wait()

    def elem_body(i, _):
        off_u = uoff_s[i]
        for j in range(EMB // 16):
            uout[i, pl.ds(16 * j, 16)] = ubuf[i, pl.ds(off_u + 16 * j, 16)]
        return 0

    lax.fori_loop(0, BPW, elem_body, 0)
    pltpu.sync_copy(
        uout, user_out.at[pl.ds(orow, BPW), pl.ds(ocol, EMB)])


_sc_user = functools.partial(
    pl.kernel,
    out_type=jax.ShapeDtypeStruct((B // 2, 2 * EMB), jnp.float32),
    mesh=plsc.VectorSubcoreMesh(core_axis_name="c", subcore_axis_name="s"),
    compiler_params=pltpu.CompilerParams(use_tc_tiling_on_sc=False),
    scratch_types=[
        pltpu.VMEM((BPW,), jnp.int32),
        pltpu.VMEM((BPW,), jnp.int32),
        pltpu.VMEM((BPW, 2 * EMB), jnp.float32),
        pltpu.VMEM((BPW, EMB), jnp.float32),
        pltpu.SMEM((BPW,), jnp.int32),
        pltpu.SemaphoreType.DMA,
    ],
)(_sc_user_body)


def _tc_body(qsum_ref, user_ref, w_ref, b_ref, out_ref):
    qp = qsum_ref[...]
    up = user_ref[...]
    q = jnp.concatenate([qp[:, :EMB], qp[:, EMB:]], axis=0) * (1.0 / QLEN)
    u = jnp.concatenate([up[:, :EMB], up[:, EMB:]], axis=0)
    z = lax.dot_general(q, w_ref[...], (((1,), (1,)), ((), ())),
                        preferred_element_type=jnp.float32)
    z = z + b_ref[...]
    out = 0.5 * jnp.tanh(z) + 0.5 * u
    out_ref[...] = out.reshape(2, out.shape[0] // 2, EMB)


def _tc_call(qsum, user_rows, w, b2d):
    blk = 1024
    return pl.pallas_call(
        _tc_body,
        grid=(B // 2 // blk,),
        in_specs=[
            pl.BlockSpec((blk, 2 * EMB), lambda i: (i, 0)),
            pl.BlockSpec((blk, 2 * EMB), lambda i: (i, 0)),
            pl.BlockSpec((EMB, EMB), lambda i: (0, 0)),
            pl.BlockSpec((1, EMB), lambda i: (0, 0)),
        ],
        out_specs=pl.BlockSpec((2, blk, EMB), lambda i: (0, i, 0)),
        out_shape=jax.ShapeDtypeStruct((2, B // 2, EMB), jnp.float32),
    )(qsum, user_rows, w, b2d)


@jax.jit
def kernel(users, items, query_words, word_embedding, entity_embedding,
           query_proj_w, query_proj_b):
    del items
    word2 = _repack(word_embedding.T, WORD)
    qwp, qwo = _qw_prep(query_words)
    qsum = _sc_word(qwp, qwo, word2)
    ent_t = lax.optimization_barrier((entity_embedding.T, word2, qwp))[0]
    ent2 = _repack(ent_t, ENT)
    user_rows = _sc_user(_pair_row(users), _pair_off(users), ent2)
    out3d = _tc_call(qsum, user_rows, query_proj_w,
                     query_proj_b.reshape(1, EMB))
    return out3d.reshape(B, EMB)


# speedup vs baseline: 1.4537x; 1.4537x over previous
import functools

import jax
import jax.numpy as jnp
from jax import lax
from jax.experimental import pallas as pl
from jax.experimental.pallas import tpu as pltpu
from jax.experimental.pallas import tpu_sc as plsc

B = 16384
EMB = 64
QLEN = 20
ENT = 1000000
WORD = 100000
NC = 2
NS = 16
NW = NC * NS
BPW = B // NW
CB = 16
NCHUNK = BPW // CB


RBLK = 2048


def _pair_row(g):
    return (g >> 11) * (RBLK // 2) + (g & (RBLK // 2 - 1))


def _pair_off(g):
    return ((g >> 10) & 1) * EMB


def _repack_body(x_ref, out_ref):
    x = x_ref[...]
    xs = jnp.concatenate([x[:, :RBLK // 2], x[:, RBLK // 2:]], axis=0)
    out_ref[...] = jnp.transpose(xs)


def _repack(table_t, n):
    grid = (n + RBLK - 1) // RBLK
    return pl.pallas_call(
        _repack_body,
        grid=(grid,),
        in_specs=[pl.BlockSpec((EMB, RBLK), lambda i: (0, i))],
        out_specs=pl.BlockSpec((RBLK // 2, 2 * EMB), lambda i: (i, 0)),
        out_shape=jax.ShapeDtypeStruct((grid * RBLK // 2, 2 * EMB),
                                       jnp.float32),
    )(table_t)


def _qw_prep_body(qw_ref, idx_ref, off_ref):
    pad = jnp.zeros((24 - QLEN, 128), jnp.int32)
    xf = lax.bitcast_convert_type(qw_ref[...], jnp.float32)
    xt = lax.bitcast_convert_type(jnp.transpose(xf), jnp.int32)
    row = _pair_row(xt)
    off = _pair_off(xt)
    for s in range(8):
        idx_ref[:, s, :] = jnp.concatenate(
            [row[:, 128 * s:128 * (s + 1)], pad], axis=0)
        off_ref[:, s, :] = jnp.concatenate(
            [off[:, 128 * s:128 * (s + 1)], pad], axis=0)


def _qw_prep(query_words):
    return pl.pallas_call(
        _qw_prep_body,
        grid=(B // 1024,),
        in_specs=[pl.BlockSpec((1024, QLEN), lambda i: (i, 0))],
        out_specs=(
            pl.BlockSpec((24, 8, 128), lambda i: (0, i, 0)),
            pl.BlockSpec((24, 8, 128), lambda i: (0, i, 0)),
        ),
        out_shape=(
            jax.ShapeDtypeStruct((24, B // 128, 128), jnp.int32),
            jax.ShapeDtypeStruct((24, B // 128, 128), jnp.int32),
        ),
    )(query_words)


def _sc_word_body(qwp_hbm, qwo_hbm, word_hbm, qsum_out,
                  widx, woff, wrows, qbuf, woff_s, sem0, sem1):
    wid = lax.axis_index("s") * NC + lax.axis_index("c")
    orow = (wid % (NW // 2)) * BPW
    ocol = (wid // (NW // 2)) * EMB
    sems = (sem0, sem1)

    pltpu.sync_copy(
        qwp_hbm.at[pl.ds(0, 24), pl.ds(wid * 4, 4)], widx)
    pltpu.sync_copy(
        qwo_hbm.at[pl.ds(0, 24), pl.ds(wid * 4, 4)], woff)

    def idx_slice(w, c):
        return widx.at[w, c // 8, pl.ds((c % 8) * CB, CB)]

    def fire(c, p):
        sem = sems[p]
        for w in range(QLEN):
            pltpu.async_copy(word_hbm.at[idx_slice(w, c)],
                             wrows.at[p].at[w], sem)

    def drain(c, p):
        sem = sems[p]
        for w in range(QLEN):
            pltpu.make_async_copy(word_hbm.at[idx_slice(w, c)],
                                  wrows.at[p].at[w], sem).wait()

    def compute(c, p):
        wr = wrows.at[p]
        qb = qbuf.at[p]
        ws = woff_s.at[p]
        for w in range(QLEN):
            wv = woff[w, c // 8, pl.ds((c % 8) * CB, CB)]
            for i in range(CB):
                ws[w, i] = wv[i]

        def elem_body(i, _):
            accs = [None] * (EMB // 16)
            for w in range(QLEN):
                off_w = ws[w, i]
                for j in range(EMB // 16):
                    v = wr[w, i, pl.ds(off_w + 16 * j, 16)]
                    accs[j] = v if w == 0 else accs[j] + v
            for j in range(EMB // 16):
                qb[i, pl.ds(16 * j, 16)] = accs[j]
            return 0

        lax.fori_loop(0, CB, elem_body, 0)
        pltpu.sync_copy(
            qb, qsum_out.at[pl.ds(orow + c * CB, CB), pl.ds(ocol, EMB)])

    fire(0, 0)

    def pair_body(t, _):
        c0 = 2 * t
        fire(c0 + 1, 1)
        drain(c0, 0)
        compute(c0, 0)

        @pl.when(t < NCHUNK // 2 - 1)
        def _():
            fire(c0 + 2, 0)

        drain(c0 + 1, 1)
        compute(c0 + 1, 1)
        return 0

    lax.fori_loop(0, NCHUNK // 2, pair_body, 0)


_sc_word = functools.partial(
    pl.kernel,
    out_type=jax.ShapeDtypeStruct((B // 2, 2 * EMB), jnp.float32),
    mesh=plsc.VectorSubcoreMesh(core_axis_name="c", subcore_axis_name="s"),
    compiler_params=pltpu.CompilerParams(use_tc_tiling_on_sc=False),
    scratch_types=[
        pltpu.VMEM((24, 4, 128), jnp.int32),
        pltpu.VMEM((24, 4, 128), jnp.int32),
        pltpu.VMEM((2, QLEN, CB, 2 * EMB), jnp.float32),
        pltpu.VMEM((2, CB, EMB), jnp.float32),
        pltpu.SMEM((2, 24, CB), jnp.int32),
        pltpu.SemaphoreType.DMA,
        pltpu.SemaphoreType.DMA,
    ],
)(_sc_word_body)


def _sc_user_body(up_hbm, uo_hbm, ent_hbm, user_out,
                  uidx, uoff, ubuf, uout, uoff_s, sem0):
    wid = lax.axis_index("s") * NC + lax.axis_index("c")
    base = wid * BPW
    orow = (wid % (NW // 2)) * BPW
    ocol = (wid // (NW // 2)) * EMB

    pltpu.sync_copy(up_hbm.at[pl.ds(base, BPW)], uidx)
    pltpu.sync_copy(uo_hbm.at[pl.ds(base, BPW)], uoff)

    cps = [pltpu.async_copy(ent_hbm.at[uidx.at[pl.ds(k * 128, 128)]],
                            ubuf.at[pl.ds(k * 128, 128)], sem0)
           for k in range(BPW // 128)]

    for cc in range(BPW // 16):
        uv = uoff[pl.ds(cc * 16, 16)]
        for i in range(16):
            uoff_s[cc * 16 + i] = uv[i]

    for cp in cps:
        cp.wait()

    def elem_body(i, _):
        off_u = uoff_s[i]
        for j in range(EMB // 16):
            uout[i, pl.ds(16 * j, 16)] = ubuf[i, pl.ds(off_u + 16 * j, 16)]
        return 0

    lax.fori_loop(0, BPW, elem_body, 0)
    pltpu.sync_copy(
        uout, user_out.at[pl.ds(orow, BPW), pl.ds(ocol, EMB)])


_sc_user = functools.partial(
    pl.kernel,
    out_type=jax.ShapeDtypeStruct((B // 2, 2 * EMB), jnp.float32),
    mesh=plsc.VectorSubcoreMesh(core_axis_name="c", subcore_axis_name="s"),
    compiler_params=pltpu.CompilerParams(use_tc_tiling_on_sc=False),
    scratch_types=[
        pltpu.VMEM((BPW,), jnp.int32),
        pltpu.VMEM((BPW,), jnp.int32),
        pltpu.VMEM((BPW, 2 * EMB), jnp.float32),
        pltpu.VMEM((BPW, EMB), jnp.float32),
        pltpu.SMEM((BPW,), jnp.int32),
        pltpu.SemaphoreType.DMA,
    ],
)(_sc_user_body)


def _tc_body(qsum_ref, user_ref, w_ref, b_ref, out_ref):
    qp = qsum_ref[...]
    up = user_ref[...]
    q = jnp.concatenate([qp[:, :EMB], qp[:, EMB:]], axis=0) * (1.0 / QLEN)
    u = jnp.concatenate([up[:, :EMB], up[:, EMB:]], axis=0)
    z = lax.dot_general(q, w_ref[...], (((1,), (1,)), ((), ())),
                        preferred_element_type=jnp.float32)
    z = z + b_ref[...]
    out = 0.5 * jnp.tanh(z) + 0.5 * u
    out_ref[...] = out.reshape(2, out.shape[0] // 2, EMB)


def _tc_call(qsum, user_rows, w, b2d):
    blk = 1024
    return pl.pallas_call(
        _tc_body,
        grid=(B // 2 // blk,),
        in_specs=[
            pl.BlockSpec((blk, 2 * EMB), lambda i: (i, 0)),
            pl.BlockSpec((blk, 2 * EMB), lambda i: (i, 0)),
            pl.BlockSpec((EMB, EMB), lambda i: (0, 0)),
            pl.BlockSpec((1, EMB), lambda i: (0, 0)),
        ],
        out_specs=pl.BlockSpec((2, blk, EMB), lambda i: (0, i, 0)),
        out_shape=jax.ShapeDtypeStruct((2, B // 2, EMB), jnp.float32),
    )(qsum, user_rows, w, b2d)


@jax.jit
def kernel(users, items, query_words, word_embedding, entity_embedding,
           query_proj_w, query_proj_b):
    del items
    word2 = _repack(word_embedding.T, WORD)
    qwp, qwo = _qw_prep(query_words)
    qsum = _sc_word(qwp, qwo, word2)
    ent_t = lax.optimization_barrier((entity_embedding.T, word2, qwp))[0]
    ent2 = _repack(ent_t, ENT)
    user_rows = _sc_user(_pair_row(users), _pair_off(users), ent2)
    out3d = _tc_call(qsum, user_rows, query_proj_w,
                     query_proj_b.reshape(1, EMB))
    return out3d.reshape(B, EMB)


# speedup vs baseline: 2.2823x; 1.5700x over previous
import functools

import jax
import jax.numpy as jnp
from jax import lax
from jax.experimental import pallas as pl
from jax.experimental.pallas import tpu as pltpu
from jax.experimental.pallas import tpu_sc as plsc

B = 16384
EMB = 64
QLEN = 20
ENT = 1000000
WORD = 100000
NC = 2
NS = 16
NW = NC * NS
BPW = B // NW
CB = 16
NCHUNK = BPW // CB


RLOG = 13
RBLK = 1 << RLOG


def _pair_row(g):
    return (g >> RLOG) * (RBLK // 2) + (g & (RBLK // 2 - 1))


def _pair_off(g):
    return ((g >> (RLOG - 1)) & 1) * EMB


def _repack_body(x_ref, out_ref):
    x = x_ref[...]
    xs = jnp.concatenate([x[:, :RBLK // 2], x[:, RBLK // 2:]], axis=0)
    out_ref[...] = jnp.transpose(xs)


def _repack(table_t, n):
    grid = (n + RBLK - 1) // RBLK
    return pl.pallas_call(
        _repack_body,
        grid=(grid,),
        in_specs=[pl.BlockSpec((EMB, RBLK), lambda i: (0, i))],
        out_specs=pl.BlockSpec((RBLK // 2, 2 * EMB), lambda i: (i, 0)),
        out_shape=jax.ShapeDtypeStruct((grid * RBLK // 2, 2 * EMB),
                                       jnp.float32),
    )(table_t)


def _qw_prep_body(qw_ref, idx_ref, off_ref):
    pad = jnp.zeros((24 - QLEN, 128), jnp.int32)
    xf = lax.bitcast_convert_type(qw_ref[...], jnp.float32)
    xf = jnp.concatenate(
        [xf, jnp.zeros((xf.shape[0], 128 - QLEN), jnp.float32)], axis=1)
    xt = lax.bitcast_convert_type(jnp.transpose(xf), jnp.int32)[:QLEN]
    row = _pair_row(xt)
    off = _pair_off(xt)
    for s in range(8):
        idx_ref[:, s, :] = jnp.concatenate(
            [row[:, 128 * s:128 * (s + 1)], pad], axis=0)
        off_ref[:, s, :] = jnp.concatenate(
            [off[:, 128 * s:128 * (s + 1)], pad], axis=0)


def _qw_prep(query_words):
    return pl.pallas_call(
        _qw_prep_body,
        grid=(B // 1024,),
        in_specs=[pl.BlockSpec((1024, QLEN), lambda i: (i, 0))],
        out_specs=(
            pl.BlockSpec((24, 8, 128), lambda i: (0, i, 0)),
            pl.BlockSpec((24, 8, 128), lambda i: (0, i, 0)),
        ),
        out_shape=(
            jax.ShapeDtypeStruct((24, B // 128, 128), jnp.int32),
            jax.ShapeDtypeStruct((24, B // 128, 128), jnp.int32),
        ),
    )(query_words)


def _sc_word_body(qwp_hbm, qwo_hbm, word_hbm, qsum_out,
                  widx, woff, wrows, qbuf, woff_s, sem0, sem1):
    wid = lax.axis_index("s") * NC + lax.axis_index("c")
    orow = (wid % (NW // 2)) * BPW
    ocol = (wid // (NW // 2)) * EMB
    sems = (sem0, sem1)

    pltpu.sync_copy(
        qwp_hbm.at[pl.ds(0, 24), pl.ds(wid * 4, 4)], widx)
    pltpu.sync_copy(
        qwo_hbm.at[pl.ds(0, 24), pl.ds(wid * 4, 4)], woff)

    def idx_slice(w, c):
        return widx.at[w, c // 8, pl.ds((c % 8) * CB, CB)]

    def fire(c, p):
        sem = sems[p]
        for w in range(QLEN):
            pltpu.async_copy(word_hbm.at[idx_slice(w, c)],
                             wrows.at[p].at[w], sem)

    def drain(c, p):
        sem = sems[p]
        for w in range(QLEN):
            pltpu.make_async_copy(word_hbm.at[idx_slice(w, c)],
                                  wrows.at[p].at[w], sem).wait()

    def compute(c, p):
        wr = wrows.at[p]
        qb = qbuf.at[p]
        ws = woff_s.at[p]
        for w in range(QLEN):
            wv = woff[w, c // 8, pl.ds((c % 8) * CB, CB)]
            for i in range(CB):
                ws[w, i] = wv[i]

        def elem_body(i, _):
            accs = [None] * (EMB // 16)
            for w in range(QLEN):
                off_w = ws[w, i]
                for j in range(EMB // 16):
                    v = wr[w, i, pl.ds(off_w + 16 * j, 16)]
                    accs[j] = v if w == 0 else accs[j] + v
            for j in range(EMB // 16):
                qb[i, pl.ds(16 * j, 16)] = accs[j]
            return 0

        lax.fori_loop(0, CB, elem_body, 0)
        pltpu.sync_copy(
            qb, qsum_out.at[pl.ds(orow + c * CB, CB), pl.ds(ocol, EMB)])

    fire(0, 0)

    def pair_body(t, _):
        c0 = 2 * t
        fire(c0 + 1, 1)
        drain(c0, 0)
        compute(c0, 0)

        @pl.when(t < NCHUNK // 2 - 1)
        def _():
            fire(c0 + 2, 0)

        drain(c0 + 1, 1)
        compute(c0 + 1, 1)
        return 0

    lax.fori_loop(0, NCHUNK // 2, pair_body, 0)


_sc_word = functools.partial(
    pl.kernel,
    out_type=jax.ShapeDtypeStruct((B // 2, 2 * EMB), jnp.float32),
    mesh=plsc.VectorSubcoreMesh(core_axis_name="c", subcore_axis_name="s"),
    compiler_params=pltpu.CompilerParams(use_tc_tiling_on_sc=False),
    scratch_types=[
        pltpu.VMEM((24, 4, 128), jnp.int32),
        pltpu.VMEM((24, 4, 128), jnp.int32),
        pltpu.VMEM((2, QLEN, CB, 2 * EMB), jnp.float32),
        pltpu.VMEM((2, CB, EMB), jnp.float32),
        pltpu.SMEM((2, 24, CB), jnp.int32),
        pltpu.SemaphoreType.DMA,
        pltpu.SemaphoreType.DMA,
    ],
)(_sc_word_body)


def _sc_user_body(up_hbm, uo_hbm, ent_hbm, user_out,
                  uidx, uoff, ubuf, uout, uoff_s, sem0):
    wid = lax.axis_index("s") * NC + lax.axis_index("c")
    base = wid * BPW
    orow = (wid % (NW // 2)) * BPW
    ocol = (wid // (NW // 2)) * EMB

    pltpu.sync_copy(up_hbm.at[pl.ds(base, BPW)], uidx)
    pltpu.sync_copy(uo_hbm.at[pl.ds(base, BPW)], uoff)

    cps = [pltpu.async_copy(ent_hbm.at[uidx.at[pl.ds(k * 128, 128)]],
                            ubuf.at[pl.ds(k * 128, 128)], sem0)
           for k in range(BPW // 128)]

    for cc in range(BPW // 16):
        uv = uoff[pl.ds(cc * 16, 16)]
        for i in range(16):
            uoff_s[cc * 16 + i] = uv[i]

    for cp in cps:
        cp.wait()

    def elem_body(i, _):
        off_u = uoff_s[i]
        for j in range(EMB // 16):
            uout[i, pl.ds(16 * j, 16)] = ubuf[i, pl.ds(off_u + 16 * j, 16)]
        return 0

    lax.fori_loop(0, BPW, elem_body, 0)
    pltpu.sync_copy(
        uout, user_out.at[pl.ds(orow, BPW), pl.ds(ocol, EMB)])


_sc_user = functools.partial(
    pl.kernel,
    out_type=jax.ShapeDtypeStruct((B // 2, 2 * EMB), jnp.float32),
    mesh=plsc.VectorSubcoreMesh(core_axis_name="c", subcore_axis_name="s"),
    compiler_params=pltpu.CompilerParams(use_tc_tiling_on_sc=False),
    scratch_types=[
        pltpu.VMEM((BPW,), jnp.int32),
        pltpu.VMEM((BPW,), jnp.int32),
        pltpu.VMEM((BPW, 2 * EMB), jnp.float32),
        pltpu.VMEM((BPW, EMB), jnp.float32),
        pltpu.SMEM((BPW,), jnp.int32),
        pltpu.SemaphoreType.DMA,
    ],
)(_sc_user_body)


def _tc_body(qsum_ref, user_ref, w_ref, b_ref, out_ref):
    qp = qsum_ref[...]
    up = user_ref[...]
    q = jnp.concatenate([qp[:, :EMB], qp[:, EMB:]], axis=0) * (1.0 / QLEN)
    u = jnp.concatenate([up[:, :EMB], up[:, EMB:]], axis=0)
    z = lax.dot_general(q, w_ref[...], (((1,), (1,)), ((), ())),
                        preferred_element_type=jnp.float32)
    z = z + b_ref[...]
    out = 0.5 * jnp.tanh(z) + 0.5 * u
    out_ref[...] = out.reshape(2, out.shape[0] // 2, EMB)


def _tc_call(qsum, user_rows, w, b2d):
    blk = 1024
    return pl.pallas_call(
        _tc_body,
        grid=(B // 2 // blk,),
        in_specs=[
            pl.BlockSpec((blk, 2 * EMB), lambda i: (i, 0)),
            pl.BlockSpec((blk, 2 * EMB), lambda i: (i, 0)),
            pl.BlockSpec((EMB, EMB), lambda i: (0, 0)),
            pl.BlockSpec((1, EMB), lambda i: (0, 0)),
        ],
        out_specs=pl.BlockSpec((2, blk, EMB), lambda i: (0, i, 0)),
        out_shape=jax.ShapeDtypeStruct((2, B // 2, EMB), jnp.float32),
    )(qsum, user_rows, w, b2d)


@jax.jit
def kernel(users, items, query_words, word_embedding, entity_embedding,
           query_proj_w, query_proj_b):
    del items
    word2 = _repack(word_embedding.T, WORD)
    qwp, qwo = _qw_prep(query_words)
    qsum = _sc_word(qwp, qwo, word2)
    ent_t = lax.optimization_barrier((entity_embedding.T, word2, qwp))[0]
    ent2 = _repack(ent_t, ENT)
    user_rows = _sc_user(_pair_row(users), _pair_off(users), ent2)
    out3d = _tc_call(qsum, user_rows, query_proj_w,
                     query_proj_b.reshape(1, EMB))
    return out3d.reshape(B, EMB)


# speedup vs baseline: 2.3791x; 1.0424x over previous
import functools

import jax
import jax.numpy as jnp
from jax import lax
from jax.experimental import pallas as pl
from jax.experimental.pallas import tpu as pltpu
from jax.experimental.pallas import tpu_sc as plsc

B = 16384
EMB = 64
QLEN = 20
ENT = 1000000
WORD = 100000
NC = 2
NS = 16
NW = NC * NS
BPW = B // NW
CB = 16
NCHUNK = BPW // CB


RLOG = 14
RBLK = 1 << RLOG


def _pair_row(g):
    return (g >> RLOG) * (RBLK // 2) + (g & (RBLK // 2 - 1))


def _pair_off(g):
    return ((g >> (RLOG - 1)) & 1) * EMB


def _repack_body(x_ref, out_ref):
    x = x_ref[...]
    xs = jnp.concatenate([x[:, :RBLK // 2], x[:, RBLK // 2:]], axis=0)
    out_ref[...] = jnp.transpose(xs)


def _repack(table_t, n):
    grid = (n + RBLK - 1) // RBLK
    return pl.pallas_call(
        _repack_body,
        grid=(grid,),
        in_specs=[pl.BlockSpec((EMB, RBLK), lambda i: (0, i))],
        out_specs=pl.BlockSpec((RBLK // 2, 2 * EMB), lambda i: (i, 0)),
        out_shape=jax.ShapeDtypeStruct((grid * RBLK // 2, 2 * EMB),
                                       jnp.float32),
    )(table_t)


def _qw_prep_body(qw_ref, idx_ref, off_ref):
    pad = jnp.zeros((24 - QLEN, 128), jnp.int32)
    xf = lax.bitcast_convert_type(qw_ref[...], jnp.float32)
    xf = jnp.concatenate(
        [xf, jnp.zeros((xf.shape[0], 128 - QLEN), jnp.float32)], axis=1)
    xt = lax.bitcast_convert_type(jnp.transpose(xf), jnp.int32)[:QLEN]
    row = _pair_row(xt)
    off = _pair_off(xt)
    for s in range(8):
        idx_ref[:, s, :] = jnp.concatenate(
            [row[:, 128 * s:128 * (s + 1)], pad], axis=0)
        off_ref[:, s, :] = jnp.concatenate(
            [off[:, 128 * s:128 * (s + 1)], pad], axis=0)


def _qw_prep(query_words):
    return pl.pallas_call(
        _qw_prep_body,
        grid=(B // 1024,),
        in_specs=[pl.BlockSpec((1024, QLEN), lambda i: (i, 0))],
        out_specs=(
            pl.BlockSpec((24, 8, 128), lambda i: (0, i, 0)),
            pl.BlockSpec((24, 8, 128), lambda i: (0, i, 0)),
        ),
        out_shape=(
            jax.ShapeDtypeStruct((24, B // 128, 128), jnp.int32),
            jax.ShapeDtypeStruct((24, B // 128, 128), jnp.int32),
        ),
    )(query_words)


def _sc_word_body(qwp_hbm, qwo_hbm, word_hbm, qsum_out,
                  widx, woff, wrows, qbuf, woff_s, sem0, sem1):
    wid = lax.axis_index("s") * NC + lax.axis_index("c")
    orow = (wid % (NW // 2)) * BPW
    ocol = (wid // (NW // 2)) * EMB
    sems = (sem0, sem1)

    pltpu.sync_copy(
        qwp_hbm.at[pl.ds(0, 24), pl.ds(wid * 4, 4)], widx)
    pltpu.sync_copy(
        qwo_hbm.at[pl.ds(0, 24), pl.ds(wid * 4, 4)], woff)

    def idx_slice(w, c):
        return widx.at[w, c // 8, pl.ds((c % 8) * CB, CB)]

    def fire(c, p):
        sem = sems[p]
        for w in range(QLEN):
            pltpu.async_copy(word_hbm.at[idx_slice(w, c)],
                             wrows.at[p].at[w], sem)

    def drain(c, p):
        sem = sems[p]
        for w in range(QLEN):
            pltpu.make_async_copy(word_hbm.at[idx_slice(w, c)],
                                  wrows.at[p].at[w], sem).wait()

    def compute(c, p):
        wr = wrows.at[p]
        qb = qbuf.at[p]
        ws = woff_s.at[p]
        for w in range(QLEN):
            wv = woff[w, c // 8, pl.ds((c % 8) * CB, CB)]
            for i in range(CB):
                ws[w, i] = wv[i]

        def elem_body(i, _):
            accs = [None] * (EMB // 16)
            for w in range(QLEN):
                off_w = ws[w, i]
                for j in range(EMB // 16):
                    v = wr[w, i, pl.ds(off_w + 16 * j, 16)]
                    accs[j] = v if w == 0 else accs[j] + v
            for j in range(EMB // 16):
                qb[i, pl.ds(16 * j, 16)] = accs[j]
            return 0

        lax.fori_loop(0, CB, elem_body, 0)
        pltpu.sync_copy(
            qb, qsum_out.at[pl.ds(orow + c * CB, CB), pl.ds(ocol, EMB)])

    fire(0, 0)

    def pair_body(t, _):
        c0 = 2 * t
        fire(c0 + 1, 1)
        drain(c0, 0)
        compute(c0, 0)

        @pl.when(t < NCHUNK // 2 - 1)
        def _():
            fire(c0 + 2, 0)

        drain(c0 + 1, 1)
        compute(c0 + 1, 1)
        return 0

    lax.fori_loop(0, NCHUNK // 2, pair_body, 0)


_sc_word = functools.partial(
    pl.kernel,
    out_type=jax.ShapeDtypeStruct((B // 2, 2 * EMB), jnp.float32),
    mesh=plsc.VectorSubcoreMesh(core_axis_name="c", subcore_axis_name="s"),
    compiler_params=pltpu.CompilerParams(use_tc_tiling_on_sc=False),
    scratch_types=[
        pltpu.VMEM((24, 4, 128), jnp.int32),
        pltpu.VMEM((24, 4, 128), jnp.int32),
        pltpu.VMEM((2, QLEN, CB, 2 * EMB), jnp.float32),
        pltpu.VMEM((2, CB, EMB), jnp.float32),
        pltpu.SMEM((2, 24, CB), jnp.int32),
        pltpu.SemaphoreType.DMA,
        pltpu.SemaphoreType.DMA,
    ],
)(_sc_word_body)


def _sc_user_body(up_hbm, uo_hbm, ent_hbm, user_out,
                  uidx, uoff, ubuf, uout, uoff_s, sem0):
    wid = lax.axis_index("s") * NC + lax.axis_index("c")
    base = wid * BPW
    orow = (wid % (NW // 2)) * BPW
    ocol = (wid // (NW // 2)) * EMB

    pltpu.sync_copy(up_hbm.at[pl.ds(base, BPW)], uidx)
    pltpu.sync_copy(uo_hbm.at[pl.ds(base, BPW)], uoff)

    cps = [pltpu.async_copy(ent_hbm.at[uidx.at[pl.ds(k * 128, 128)]],
                            ubuf.at[pl.ds(k * 128, 128)], sem0)
           for k in range(BPW // 128)]

    for cc in range(BPW // 16):
        uv = uoff[pl.ds(cc * 16, 16)]
        for i in range(16):
            uoff_s[cc * 16 + i] = uv[i]

    for cp in cps:
        cp.wait()

    def elem_body(i, _):
        off_u = uoff_s[i]
        for j in range(EMB // 16):
            uout[i, pl.ds(16 * j, 16)] = ubuf[i, pl.ds(off_u + 16 * j, 16)]
        return 0

    lax.fori_loop(0, BPW, elem_body, 0)
    pltpu.sync_copy(
        uout, user_out.at[pl.ds(orow, BPW), pl.ds(ocol, EMB)])


_sc_user = functools.partial(
    pl.kernel,
    out_type=jax.ShapeDtypeStruct((B // 2, 2 * EMB), jnp.float32),
    mesh=plsc.VectorSubcoreMesh(core_axis_name="c", subcore_axis_name="s"),
    compiler_params=pltpu.CompilerParams(use_tc_tiling_on_sc=False),
    scratch_types=[
        pltpu.VMEM((BPW,), jnp.int32),
        pltpu.VMEM((BPW,), jnp.int32),
        pltpu.VMEM((BPW, 2 * EMB), jnp.float32),
        pltpu.VMEM((BPW, EMB), jnp.float32),
        pltpu.SMEM((BPW,), jnp.int32),
        pltpu.SemaphoreType.DMA,
    ],
)(_sc_user_body)


def _tc_body(qsum_ref, user_ref, w_ref, b_ref, out_ref):
    qp = qsum_ref[...]
    up = user_ref[...]
    q = jnp.concatenate([qp[:, :EMB], qp[:, EMB:]], axis=0) * (1.0 / QLEN)
    u = jnp.concatenate([up[:, :EMB], up[:, EMB:]], axis=0)
    z = lax.dot_general(q, w_ref[...], (((1,), (1,)), ((), ())),
                        preferred_element_type=jnp.float32)
    z = z + b_ref[...]
    out = 0.5 * jnp.tanh(z) + 0.5 * u
    out_ref[...] = out.reshape(2, out.shape[0] // 2, EMB)


def _tc_call(qsum, user_rows, w, b2d):
    blk = 1024
    return pl.pallas_call(
        _tc_body,
        grid=(B // 2 // blk,),
        in_specs=[
            pl.BlockSpec((blk, 2 * EMB), lambda i: (i, 0)),
            pl.BlockSpec((blk, 2 * EMB), lambda i: (i, 0)),
            pl.BlockSpec((EMB, EMB), lambda i: (0, 0)),
            pl.BlockSpec((1, EMB), lambda i: (0, 0)),
        ],
        out_specs=pl.BlockSpec((2, blk, EMB), lambda i: (0, i, 0)),
        out_shape=jax.ShapeDtypeStruct((2, B // 2, EMB), jnp.float32),
    )(qsum, user_rows, w, b2d)


@jax.jit
def kernel(users, items, query_words, word_embedding, entity_embedding,
           query_proj_w, query_proj_b):
    del items
    word2 = _repack(word_embedding.T, WORD)
    qwp, qwo = _qw_prep(query_words)
    qsum = _sc_word(qwp, qwo, word2)
    ent_t = lax.optimization_barrier((entity_embedding.T, word2, qwp))[0]
    ent2 = _repack(ent_t, ENT)
    user_rows = _sc_user(_pair_row(users), _pair_off(users), ent2)
    out3d = _tc_call(qsum, user_rows, query_proj_w,
                     query_proj_b.reshape(1, EMB))
    return out3d.reshape(B, EMB)


# speedup vs baseline: 2.3981x; 1.0080x over previous
import functools

import jax
import jax.numpy as jnp
from jax import lax
from jax.experimental import pallas as pl
from jax.experimental.pallas import tpu as pltpu
from jax.experimental.pallas import tpu_sc as plsc

B = 16384
EMB = 64
QLEN = 20
ENT = 1000000
WORD = 100000
NC = 2
NS = 16
NW = NC * NS
BPW = B // NW
CB = 16
NCHUNK = BPW // CB


RLOG = 15
RBLK = 1 << RLOG


def _pair_row(g):
    return (g >> RLOG) * (RBLK // 2) + (g & (RBLK // 2 - 1))


def _pair_off(g):
    return ((g >> (RLOG - 1)) & 1) * EMB


def _repack_body(x_ref, out_ref):
    x = x_ref[...]
    xs = jnp.concatenate([x[:, :RBLK // 2], x[:, RBLK // 2:]], axis=0)
    out_ref[...] = jnp.transpose(xs)


def _repack(table_t, n):
    grid = (n + RBLK - 1) // RBLK
    return pl.pallas_call(
        _repack_body,
        grid=(grid,),
        in_specs=[pl.BlockSpec((EMB, RBLK), lambda i: (0, i))],
        out_specs=pl.BlockSpec((RBLK // 2, 2 * EMB), lambda i: (i, 0)),
        out_shape=jax.ShapeDtypeStruct((grid * RBLK // 2, 2 * EMB),
                                       jnp.float32),
    )(table_t)


def _qw_prep_body(qw_ref, idx_ref, off_ref):
    pad = jnp.zeros((24 - QLEN, 128), jnp.int32)
    xf = lax.bitcast_convert_type(qw_ref[...], jnp.float32)
    xf = jnp.concatenate(
        [xf, jnp.zeros((xf.shape[0], 128 - QLEN), jnp.float32)], axis=1)
    xt = lax.bitcast_convert_type(jnp.transpose(xf), jnp.int32)[:QLEN]
    row = _pair_row(xt)
    off = _pair_off(xt)
    for s in range(8):
        idx_ref[:, s, :] = jnp.concatenate(
            [row[:, 128 * s:128 * (s + 1)], pad], axis=0)
        off_ref[:, s, :] = jnp.concatenate(
            [off[:, 128 * s:128 * (s + 1)], pad], axis=0)


def _qw_prep(query_words):
    return pl.pallas_call(
        _qw_prep_body,
        grid=(B // 1024,),
        in_specs=[pl.BlockSpec((1024, QLEN), lambda i: (i, 0))],
        out_specs=(
            pl.BlockSpec((24, 8, 128), lambda i: (0, i, 0)),
            pl.BlockSpec((24, 8, 128), lambda i: (0, i, 0)),
        ),
        out_shape=(
            jax.ShapeDtypeStruct((24, B // 128, 128), jnp.int32),
            jax.ShapeDtypeStruct((24, B // 128, 128), jnp.int32),
        ),
    )(query_words)


def _sc_word_body(qwp_hbm, qwo_hbm, word_hbm, qsum_out,
                  widx, woff, wrows, qbuf, woff_s, sem0, sem1):
    wid = lax.axis_index("s") * NC + lax.axis_index("c")
    orow = (wid % (NW // 2)) * BPW
    ocol = (wid // (NW // 2)) * EMB
    sems = (sem0, sem1)

    pltpu.sync_copy(
        qwp_hbm.at[pl.ds(0, 24), pl.ds(wid * 4, 4)], widx)
    pltpu.sync_copy(
        qwo_hbm.at[pl.ds(0, 24), pl.ds(wid * 4, 4)], woff)

    def idx_slice(w, c):
        return widx.at[w, c // 8, pl.ds((c % 8) * CB, CB)]

    def fire(c, p):
        sem = sems[p]
        for w in range(QLEN):
            pltpu.async_copy(word_hbm.at[idx_slice(w, c)],
                             wrows.at[p].at[w], sem)

    def drain(c, p):
        sem = sems[p]
        for w in range(QLEN):
            pltpu.make_async_copy(word_hbm.at[idx_slice(w, c)],
                                  wrows.at[p].at[w], sem).wait()

    def compute(c, p):
        wr = wrows.at[p]
        qb = qbuf.at[p]
        ws = woff_s.at[p]
        for w in range(QLEN):
            wv = woff[w, c // 8, pl.ds((c % 8) * CB, CB)]
            for i in range(CB):
                ws[w, i] = wv[i]

        def elem_body(i, _):
            accs = [None] * (EMB // 16)
            for w in range(QLEN):
                off_w = ws[w, i]
                for j in range(EMB // 16):
                    v = wr[w, i, pl.ds(off_w + 16 * j, 16)]
                    accs[j] = v if w == 0 else accs[j] + v
            for j in range(EMB // 16):
                qb[i, pl.ds(16 * j, 16)] = accs[j]
            return 0

        lax.fori_loop(0, CB, elem_body, 0)
        pltpu.sync_copy(
            qb, qsum_out.at[pl.ds(orow + c * CB, CB), pl.ds(ocol, EMB)])

    fire(0, 0)

    def pair_body(t, _):
        c0 = 2 * t
        fire(c0 + 1, 1)
        drain(c0, 0)
        compute(c0, 0)

        @pl.when(t < NCHUNK // 2 - 1)
        def _():
            fire(c0 + 2, 0)

        drain(c0 + 1, 1)
        compute(c0 + 1, 1)
        return 0

    lax.fori_loop(0, NCHUNK // 2, pair_body, 0)


_sc_word = functools.partial(
    pl.kernel,
    out_type=jax.ShapeDtypeStruct((B // 2, 2 * EMB), jnp.float32),
    mesh=plsc.VectorSubcoreMesh(core_axis_name="c", subcore_axis_name="s"),
    compiler_params=pltpu.CompilerParams(use_tc_tiling_on_sc=False),
    scratch_types=[
        pltpu.VMEM((24, 4, 128), jnp.int32),
        pltpu.VMEM((24, 4, 128), jnp.int32),
        pltpu.VMEM((2, QLEN, CB, 2 * EMB), jnp.float32),
        pltpu.VMEM((2, CB, EMB), jnp.float32),
        pltpu.SMEM((2, 24, CB), jnp.int32),
        pltpu.SemaphoreType.DMA,
        pltpu.SemaphoreType.DMA,
    ],
)(_sc_word_body)


def _sc_user_body(up_hbm, uo_hbm, ent_hbm, user_out,
                  uidx, uoff, ubuf, uout, uoff_s, sem0):
    wid = lax.axis_index("s") * NC + lax.axis_index("c")
    base = wid * BPW
    orow = (wid % (NW // 2)) * BPW
    ocol = (wid // (NW // 2)) * EMB

    pltpu.sync_copy(up_hbm.at[pl.ds(base, BPW)], uidx)
    pltpu.sync_copy(uo_hbm.at[pl.ds(base, BPW)], uoff)

    cps = [pltpu.async_copy(ent_hbm.at[uidx.at[pl.ds(k * 128, 128)]],
                            ubuf.at[pl.ds(k * 128, 128)], sem0)
           for k in range(BPW // 128)]

    for cc in range(BPW // 16):
        uv = uoff[pl.ds(cc * 16, 16)]
        for i in range(16):
            uoff_s[cc * 16 + i] = uv[i]

    for cp in cps:
        cp.wait()

    def elem_body(i, _):
        off_u = uoff_s[i]
        for j in range(EMB // 16):
            uout[i, pl.ds(16 * j, 16)] = ubuf[i, pl.ds(off_u + 16 * j, 16)]
        return 0

    lax.fori_loop(0, BPW, elem_body, 0)
    pltpu.sync_copy(
        uout, user_out.at[pl.ds(orow, BPW), pl.ds(ocol, EMB)])


_sc_user = functools.partial(
    pl.kernel,
    out_type=jax.ShapeDtypeStruct((B // 2, 2 * EMB), jnp.float32),
    mesh=plsc.VectorSubcoreMesh(core_axis_name="c", subcore_axis_name="s"),
    compiler_params=pltpu.CompilerParams(use_tc_tiling_on_sc=False),
    scratch_types=[
        pltpu.VMEM((BPW,), jnp.int32),
        pltpu.VMEM((BPW,), jnp.int32),
        pltpu.VMEM((BPW, 2 * EMB), jnp.float32),
        pltpu.VMEM((BPW, EMB), jnp.float32),
        pltpu.SMEM((BPW,), jnp.int32),
        pltpu.SemaphoreType.DMA,
    ],
)(_sc_user_body)


def _tc_body(qsum_ref, user_ref, w_ref, b_ref, out_ref):
    qp = qsum_ref[...]
    up = user_ref[...]
    q = jnp.concatenate([qp[:, :EMB], qp[:, EMB:]], axis=0) * (1.0 / QLEN)
    u = jnp.concatenate([up[:, :EMB], up[:, EMB:]], axis=0)
    z = lax.dot_general(q, w_ref[...], (((1,), (1,)), ((), ())),
                        preferred_element_type=jnp.float32)
    z = z + b_ref[...]
    out = 0.5 * jnp.tanh(z) + 0.5 * u
    out_ref[...] = out.reshape(2, out.shape[0] // 2, EMB)


def _tc_call(qsum, user_rows, w, b2d):
    blk = 1024
    return pl.pallas_call(
        _tc_body,
        grid=(B // 2 // blk,),
        in_specs=[
            pl.BlockSpec((blk, 2 * EMB), lambda i: (i, 0)),
            pl.BlockSpec((blk, 2 * EMB), lambda i: (i, 0)),
            pl.BlockSpec((EMB, EMB), lambda i: (0, 0)),
            pl.BlockSpec((1, EMB), lambda i: (0, 0)),
        ],
        out_specs=pl.BlockSpec((2, blk, EMB), lambda i: (0, i, 0)),
        out_shape=jax.ShapeDtypeStruct((2, B // 2, EMB), jnp.float32),
    )(qsum, user_rows, w, b2d)


@jax.jit
def kernel(users, items, query_words, word_embedding, entity_embedding,
           query_proj_w, query_proj_b):
    del items
    word2 = _repack(word_embedding.T, WORD)
    qwp, qwo = _qw_prep(query_words)
    qsum = _sc_word(qwp, qwo, word2)
    ent_t = lax.optimization_barrier((entity_embedding.T, word2, qwp))[0]
    ent2 = _repack(ent_t, ENT)
    user_rows = _sc_user(_pair_row(users), _pair_off(users), ent2)
    out3d = _tc_call(qsum, user_rows, query_proj_w,
                     query_proj_b.reshape(1, EMB))
    return out3d.reshape(B, EMB)
